# Initial kernel scaffold; baseline (speedup 1.0000x reference)
import jax
import jax.numpy as jnp
from jax.experimental import pallas as pl


def kernel(x, edge_index, edge_attr, Wl0, Wr0, b0, Wl1, Wr1, b1, Wl2, Wr2, b2):
    raise NotImplementedError("write your pallas kernel here")


# speedup vs baseline: 5.2928x; 5.2928x over previous
import functools

import jax
import jax.numpy as jnp
from jax import lax
from jax.experimental import pallas as pl
from jax.experimental.pallas import tpu as pltpu
from jax.experimental.pallas import tpu_sc as plsc

_N = 10000
_E = 320000
_D = 128
_NC = 2
_NS = 16
_NW = _NC * _NS
_EPW = _E // _NW
_K = 80
_NCHUNK = _EPW // _K
_NP = 10240
_RPT = _NP // _NS


def _zero_rows(ref, nrows, width16):
    z = jnp.zeros((16,), jnp.float32)

    def body(r, _):
        for j in range(width16):
            ref[r, pl.ds(j * 16, 16)] = z
        return 0

    lax.fori_loop(0, nrows, body, 0, unroll=False)


def _sc_edge_pass(second):
    mesh = plsc.VectorSubcoreMesh(core_axis_name="c", subcore_axis_name="s")

    out_type = (
        jax.ShapeDtypeStruct((_NC, _NP, _D), jnp.float32),
        jax.ShapeDtypeStruct((_NC, _NP), jnp.float32),
    )
    scratch = [
        pltpu.VMEM((_K,), jnp.int32),
        pltpu.VMEM((_K,), jnp.int32),
        pltpu.VMEM((_K,), jnp.float32),
        pltpu.VMEM((_K, _D), jnp.float32),
        pltpu.VMEM((_K,), jnp.float32),
        pltpu.VMEM((_RPT,), jnp.float32),
        pltpu.VMEM_SHARED((_NP, _D), jnp.float32),
        pltpu.VMEM_SHARED((_NP,), jnp.float32),
        pltpu.SemaphoreType.DMA,
        pltpu.SemaphoreType.DMA,
    ]

    def body(*refs):
        if second:
            (x_hbm, src_hbm, dst_hbm, w_hbm, inv_hbm,
             num_out, sca_out,
             src_v, dst_v, w_v, rows_v, val_v, zvec_v,
             acc_sh, sca_sh, sem, sem2) = refs
        else:
            (x_hbm, src_hbm, dst_hbm, w_hbm,
             num_out, sca_out,
             src_v, dst_v, w_v, rows_v, val_v, zvec_v,
             acc_sh, sca_sh, sem, sem2) = refs

        c = lax.axis_index("c")
        s = lax.axis_index("s")
        wid = s * _NC + c

        _zero_rows(rows_v, _K, _D // 16)
        for q in range(_RPT // _K):
            pltpu.sync_copy(rows_v, acc_sh.at[pl.ds(s * _RPT + q * _K, _K)])

        z16 = jnp.zeros((16,), jnp.float32)

        def zvec_body(i, _):
            zvec_v[pl.ds(i * 16, 16)] = z16
            return 0

        lax.fori_loop(0, _RPT // 16, zvec_body, 0, unroll=False)
        pltpu.sync_copy(zvec_v, sca_sh.at[pl.ds(s * _RPT, _RPT)])

        plsc.subcore_barrier()

        ebase = wid * _EPW

        def chunk(g, _):
            base = ebase + g * _K
            pltpu.sync_copy(src_hbm.at[pl.ds(base, _K)], src_v)
            pltpu.sync_copy(dst_hbm.at[pl.ds(base, _K)], dst_v)
            pltpu.sync_copy(w_hbm.at[pl.ds(base, _K)], w_v)
            pltpu.async_copy(x_hbm.at[src_v], rows_v, sem).wait()

            def scale(r, _):
                wv = plsc.load_gather(w_v, [jnp.full((16,), r, jnp.int32)])
                for j in range(_D // 16):
                    rows_v[r, pl.ds(j * 16, 16)] = (
                        rows_v[r, pl.ds(j * 16, 16)] * wv)
                return 0

            lax.fori_loop(0, _K, scale, 0, unroll=False)
            pltpu.sync_copy(rows_v, acc_sh.at[dst_v], add=True)

            if second:
                pltpu.async_copy(inv_hbm.at[dst_v], val_v, sem2).wait()
                for t in range(_K // 16):
                    val_v[pl.ds(t * 16, 16)] = (
                        val_v[pl.ds(t * 16, 16)] * w_v[pl.ds(t * 16, 16)])
                pltpu.sync_copy(val_v, sca_sh.at[src_v], add=True)
            else:
                pltpu.sync_copy(w_v, sca_sh.at[dst_v], add=True)
            return 0

        lax.fori_loop(0, _NCHUNK, chunk, 0, unroll=False)

        plsc.subcore_barrier()

        pltpu.sync_copy(acc_sh.at[pl.ds(s * _RPT, _RPT)],
                        num_out.at[c, pl.ds(s * _RPT, _RPT)])
        pltpu.sync_copy(sca_sh.at[pl.ds(s * _RPT, _RPT)],
                        sca_out.at[c, pl.ds(s * _RPT, _RPT)])

    return pl.kernel(body, out_type=out_type, mesh=mesh,
                     scratch_types=scratch,
                     compiler_params=pltpu.CompilerParams(
                         needs_layout_passes=False))


_sc_pass1 = _sc_edge_pass(second=False)
_sc_pass2 = _sc_edge_pass(second=True)


def _tc0_body(num_ref, den_ref, x_ref, wl_ref, wr_ref, b_ref, h1_ref, inv_ref):
    den = den_ref[0, :_N] + den_ref[1, :_N]
    inv = 1.0 / jnp.clip(den, 1e-12, None)
    num = num_ref[0, :_N] + num_ref[1, :_N]
    mean = num * inv[:, None]
    h = (jnp.dot(mean, wl_ref[...], preferred_element_type=jnp.float32)
         + jnp.dot(x_ref[...], wr_ref[...], preferred_element_type=jnp.float32)
         + b_ref[...][None, :])
    h1_ref[...] = jnp.maximum(h, 0.0)
    inv_ref[...] = inv


_tc_layer0 = pl.pallas_call(
    _tc0_body,
    out_shape=(
        jax.ShapeDtypeStruct((_N, _D), jnp.float32),
        jax.ShapeDtypeStruct((_N,), jnp.float32),
    ),
)


def _tc12_body(num_ref, inv_ref, h1_ref, c_ref, wl1_ref, wr1_ref, b1_ref,
               wl2_ref, wr2_ref, b2_ref, out_ref):
    inv = inv_ref[...]
    num = num_ref[0, :_N] + num_ref[1, :_N]
    mean = num * inv[:, None]
    h2 = (jnp.dot(mean, wl1_ref[...], preferred_element_type=jnp.float32)
          + jnp.dot(h1_ref[...], wr1_ref[...], preferred_element_type=jnp.float32)
          + b1_ref[...][None, :])
    h2 = jnp.maximum(h2, 0.0)
    cvec = (c_ref[0, :_N] + c_ref[1, :_N]) * (1.0 / _N)
    s_c = jnp.dot(cvec[None, :], h2, preferred_element_type=jnp.float32)
    s_m = jnp.sum(h2, axis=0, keepdims=True) * (1.0 / _N)
    out = (jnp.dot(s_c, wl2_ref[...], preferred_element_type=jnp.float32)
           + jnp.dot(s_m, wr2_ref[...], preferred_element_type=jnp.float32)
           + b2_ref[...][None, :])
    out_ref[...] = out


_tc_layer12 = pl.pallas_call(
    _tc12_body,
    out_shape=jax.ShapeDtypeStruct((1, 16), jnp.float32),
)


@jax.jit
def kernel(x, edge_index, edge_attr, Wl0, Wr0, b0, Wl1, Wr1, b1, Wl2, Wr2, b2):
    src = edge_index[0]
    dst = edge_index[1]
    w = edge_attr

    num0p, den0p = _sc_pass1(x, src, dst, w)
    h1, inv_den = _tc_layer0(num0p, den0p, x, Wl0, Wr0, b0)
    num1p, cp = _sc_pass2(h1, src, dst, w, inv_den)
    out = _tc_layer12(num1p, inv_den, h1, cp, Wl1, Wr1, b1, Wl2, Wr2, b2)
    return out[0]


# speedup vs baseline: 7.9488x; 1.5018x over previous
import functools

import jax
import jax.numpy as jnp
from jax import lax
from jax.experimental import pallas as pl
from jax.experimental.pallas import tpu as pltpu
from jax.experimental.pallas import tpu_sc as plsc

_N = 10000
_E = 320000
_D = 128
_NC = 2
_NS = 16
_NW = _NC * _NS
_EPW = _E // _NW
_K = 80
_NCHUNK = _EPW // _K
_NP = 10240
_RPT = _NP // _NS


def _zero_rows(ref, nrows, width16):
    z = jnp.zeros((16,), jnp.float32)

    def body(r, _):
        for j in range(width16):
            ref[r, pl.ds(j * 16, 16)] = z
        return 0

    lax.fori_loop(0, nrows, body, 0, unroll=False)


def _sc_edge_pass(second):
    mesh = plsc.VectorSubcoreMesh(core_axis_name="c", subcore_axis_name="s")

    out_type = (
        jax.ShapeDtypeStruct((_NC, _NP, _D), jnp.float32),
        jax.ShapeDtypeStruct((_NC, _NP), jnp.float32),
    )
    buf = lambda dt: pltpu.VMEM((_K,), dt)
    scratch = [
        buf(jnp.int32), buf(jnp.int32),
        buf(jnp.int32), buf(jnp.int32),
        buf(jnp.float32), buf(jnp.float32),
        pltpu.VMEM((_K, _D), jnp.float32),
        pltpu.VMEM((_K, _D), jnp.float32),
        buf(jnp.float32), buf(jnp.float32),
        pltpu.VMEM((_RPT,), jnp.float32),
        pltpu.VMEM_SHARED((_NP, _D), jnp.float32),
        pltpu.VMEM_SHARED((_NP,), jnp.float32),
        pltpu.SemaphoreType.DMA, pltpu.SemaphoreType.DMA,
        pltpu.SemaphoreType.DMA, pltpu.SemaphoreType.DMA,
    ]

    def body(*refs):
        if second:
            (x_hbm, src_hbm, dst_hbm, w_hbm, inv_hbm,
             num_out, sca_out,
             srcA, srcB, dstA, dstB, wA, wB, rowsA, rowsB, valA, valB,
             zvec_v, acc_sh, sca_sh, semA, semB, sem2A, sem2B) = refs
        else:
            (x_hbm, src_hbm, dst_hbm, w_hbm,
             num_out, sca_out,
             srcA, srcB, dstA, dstB, wA, wB, rowsA, rowsB, valA, valB,
             zvec_v, acc_sh, sca_sh, semA, semB, sem2A, sem2B) = refs

        c = lax.axis_index("c")
        s = lax.axis_index("s")
        wid = s * _NC + c

        _zero_rows(rowsA, _K, _D // 16)
        for q in range(_RPT // _K):
            pltpu.sync_copy(rowsA, acc_sh.at[pl.ds(s * _RPT + q * _K, _K)])

        z16 = jnp.zeros((16,), jnp.float32)

        def zvec_body(i, _):
            zvec_v[pl.ds(i * 16, 16)] = z16
            return 0

        lax.fori_loop(0, _RPT // 16, zvec_body, 0, unroll=False)
        pltpu.sync_copy(zvec_v, sca_sh.at[pl.ds(s * _RPT, _RPT)])

        plsc.subcore_barrier()

        ebase = wid * _EPW
        bufA = (srcA, dstA, wA, rowsA, valA, semA, sem2A)
        bufB = (srcB, dstB, wB, rowsB, valB, semB, sem2B)

        def issue(g, src_b, dst_b, w_b, rows_b, val_b, sem_b, sem2_b):
            base = ebase + g * _K
            pltpu.sync_copy(src_hbm.at[pl.ds(base, _K)], src_b)
            pltpu.sync_copy(dst_hbm.at[pl.ds(base, _K)], dst_b)
            pltpu.sync_copy(w_hbm.at[pl.ds(base, _K)], w_b)
            pltpu.async_copy(x_hbm.at[src_b], rows_b, sem_b)
            if second:
                pltpu.async_copy(inv_hbm.at[dst_b], val_b, sem2_b)

        def process(src_b, dst_b, w_b, rows_b, val_b, sem_b, sem2_b):
            pltpu.make_async_copy(x_hbm.at[src_b], rows_b, sem_b).wait()

            def scale(r, _):
                wv = plsc.load_gather(w_b, [jnp.full((16,), r, jnp.int32)])
                for j in range(_D // 16):
                    rows_b[r, pl.ds(j * 16, 16)] = (
                        rows_b[r, pl.ds(j * 16, 16)] * wv)
                return 0

            lax.fori_loop(0, _K, scale, 0, unroll=2)
            pltpu.sync_copy(rows_b, acc_sh.at[dst_b], add=True)

            if second:
                pltpu.make_async_copy(inv_hbm.at[dst_b], val_b, sem2_b).wait()
                for t in range(_K // 16):
                    val_b[pl.ds(t * 16, 16)] = (
                        val_b[pl.ds(t * 16, 16)] * w_b[pl.ds(t * 16, 16)])
                pltpu.sync_copy(val_b, sca_sh.at[src_b], add=True)
            else:
                pltpu.sync_copy(w_b, sca_sh.at[dst_b], add=True)

        assert _NCHUNK % 2 == 1
        issue(0, *bufA)

        def pair(p, _):
            g0 = 2 * p
            issue(g0 + 1, *bufB)
            process(*bufA)
            issue(g0 + 2, *bufA)
            process(*bufB)
            return 0

        lax.fori_loop(0, _NCHUNK // 2, pair, 0, unroll=False)
        process(*bufA)

        plsc.subcore_barrier()

        pltpu.sync_copy(acc_sh.at[pl.ds(s * _RPT, _RPT)],
                        num_out.at[c, pl.ds(s * _RPT, _RPT)])
        pltpu.sync_copy(sca_sh.at[pl.ds(s * _RPT, _RPT)],
                        sca_out.at[c, pl.ds(s * _RPT, _RPT)])

    return pl.kernel(body, out_type=out_type, mesh=mesh,
                     scratch_types=scratch,
                     compiler_params=pltpu.CompilerParams(
                         needs_layout_passes=False))


_sc_pass1 = _sc_edge_pass(second=False)
_sc_pass2 = _sc_edge_pass(second=True)


def _tc0_body(num_ref, den_ref, x_ref, wl_ref, wr_ref, b_ref, h1_ref, inv_ref):
    den = den_ref[0, :_N] + den_ref[1, :_N]
    inv = 1.0 / jnp.clip(den, 1e-12, None)
    num = num_ref[0, :_N] + num_ref[1, :_N]
    mean = num * inv[:, None]
    h = (jnp.dot(mean, wl_ref[...], preferred_element_type=jnp.float32)
         + jnp.dot(x_ref[...], wr_ref[...], preferred_element_type=jnp.float32)
         + b_ref[...][None, :])
    h1_ref[...] = jnp.maximum(h, 0.0)
    inv_ref[...] = inv


_tc_layer0 = pl.pallas_call(
    _tc0_body,
    out_shape=(
        jax.ShapeDtypeStruct((_N, _D), jnp.float32),
        jax.ShapeDtypeStruct((_N,), jnp.float32),
    ),
)


def _tc12_body(num_ref, inv_ref, h1_ref, c_ref, wl1_ref, wr1_ref, b1_ref,
               wl2_ref, wr2_ref, b2_ref, out_ref):
    inv = inv_ref[...]
    num = num_ref[0, :_N] + num_ref[1, :_N]
    mean = num * inv[:, None]
    h2 = (jnp.dot(mean, wl1_ref[...], preferred_element_type=jnp.float32)
          + jnp.dot(h1_ref[...], wr1_ref[...], preferred_element_type=jnp.float32)
          + b1_ref[...][None, :])
    h2 = jnp.maximum(h2, 0.0)
    cvec = (c_ref[0, :_N] + c_ref[1, :_N]) * (1.0 / _N)
    s_c = jnp.dot(cvec[None, :], h2, preferred_element_type=jnp.float32)
    s_m = jnp.sum(h2, axis=0, keepdims=True) * (1.0 / _N)
    out = (jnp.dot(s_c, wl2_ref[...], preferred_element_type=jnp.float32)
           + jnp.dot(s_m, wr2_ref[...], preferred_element_type=jnp.float32)
           + b2_ref[...][None, :])
    out_ref[...] = out


_tc_layer12 = pl.pallas_call(
    _tc12_body,
    out_shape=jax.ShapeDtypeStruct((1, 16), jnp.float32),
)


@jax.jit
def kernel(x, edge_index, edge_attr, Wl0, Wr0, b0, Wl1, Wr1, b1, Wl2, Wr2, b2):
    src = edge_index[0]
    dst = edge_index[1]
    w = edge_attr

    num0p, den0p = _sc_pass1(x, src, dst, w)
    h1, inv_den = _tc_layer0(num0p, den0p, x, Wl0, Wr0, b0)
    num1p, cp = _sc_pass2(h1, src, dst, w, inv_den)
    out = _tc_layer12(num1p, inv_den, h1, cp, Wl1, Wr1, b1, Wl2, Wr2, b2)
    return out[0]


# speedup vs baseline: 12.8920x; 1.6219x over previous
import functools

import jax
import jax.numpy as jnp
from jax import lax
from jax.experimental import pallas as pl
from jax.experimental.pallas import tpu as pltpu
from jax.experimental.pallas import tpu_sc as plsc

_N = 10000
_E = 320000
_D = 128
_NC = 2
_NS = 16
_NW = _NC * _NS
_EPW = _E // _NW
_K = 80
_B = 2000
_CPB = _B // _K
_NBLK = _EPW // _B
_NP = 10240
_RPT = _NP // _NS


def _zero_rows(ref, nrows, width16):
    z = jnp.zeros((16,), jnp.float32)

    def body(r, _):
        for j in range(width16):
            ref[r, pl.ds(j * 16, 16)] = z
        return 0

    lax.fori_loop(0, nrows, body, 0, unroll=False)


def _sc_edge_pass(second):
    mesh = plsc.VectorSubcoreMesh(core_axis_name="c", subcore_axis_name="s")

    out_type = (
        jax.ShapeDtypeStruct((_NC, _NP, _D), jnp.float32),
        jax.ShapeDtypeStruct((_NC, _NP), jnp.float32),
    )
    buf = lambda dt: pltpu.VMEM((_K,), dt)
    scratch = [
        pltpu.VMEM((_B,), jnp.int32),
        pltpu.VMEM((_B,), jnp.int32),
        pltpu.VMEM((_B,), jnp.float32),
        buf(jnp.int32),
        buf(jnp.int32),
        buf(jnp.float32),
        pltpu.VMEM((_K, _D), jnp.float32),
        pltpu.VMEM((_K, _D), jnp.float32),
        buf(jnp.float32), buf(jnp.float32),
        pltpu.VMEM((_RPT,), jnp.float32),
        pltpu.VMEM_SHARED((_NP, _D), jnp.float32),
        pltpu.VMEM_SHARED((_NP,), jnp.float32),
        pltpu.SemaphoreType.DMA, pltpu.SemaphoreType.DMA,
        pltpu.SemaphoreType.DMA, pltpu.SemaphoreType.DMA,
    ]

    def body(*refs):
        if second:
            (x_hbm, src_hbm, dst_hbm, w_hbm, inv_hbm,
             num_out, sca_out,
             src_blk, dst_blk, w_blk, dstC, srcC, wC, rowsA, rowsB,
             valA, valB, zvec_v, acc_sh, sca_sh,
             semA, semB, sem2A, sem2B) = refs
        else:
            (x_hbm, src_hbm, dst_hbm, w_hbm,
             num_out, sca_out,
             src_blk, dst_blk, w_blk, dstC, srcC, wC, rowsA, rowsB,
             valA, valB, zvec_v, acc_sh, sca_sh,
             semA, semB, sem2A, sem2B) = refs

        c = lax.axis_index("c")
        s = lax.axis_index("s")
        wid = s * _NC + c
        iota = lax.iota(jnp.int32, 16)

        _zero_rows(rowsA, _K, _D // 16)
        for q in range(_RPT // _K):
            pltpu.sync_copy(rowsA, acc_sh.at[pl.ds(s * _RPT + q * _K, _K)])

        z16 = jnp.zeros((16,), jnp.float32)

        def zvec_body(i, _):
            zvec_v[pl.ds(i * 16, 16)] = z16
            return 0

        lax.fori_loop(0, _RPT // 16, zvec_body, 0, unroll=False)
        pltpu.sync_copy(zvec_v, sca_sh.at[pl.ds(s * _RPT, _RPT)])

        plsc.subcore_barrier()

        ebase = wid * _EPW
        bufA = (rowsA, valA, semA, sem2A)
        bufB = (rowsB, valB, semB, sem2B)

        def _align8(off):
            return off if isinstance(off, int) else pl.multiple_of(off, 8)

        def issue(off, rows_b, val_b, sem_b, sem2_b):
            off = _align8(off)
            pltpu.async_copy(
                x_hbm.at[src_blk.at[pl.ds(off, _K)]], rows_b, sem_b)
            if second:
                pltpu.async_copy(
                    inv_hbm.at[dst_blk.at[pl.ds(off, _K)]], val_b, sem2_b)

        def process(off, rows_b, val_b, sem_b, sem2_b):
            off = _align8(off)
            for j in range(_K // 16):
                ix = off + j * 16 + iota
                dstC[pl.ds(j * 16, 16)] = plsc.load_gather(dst_blk, [ix])
                wC[pl.ds(j * 16, 16)] = plsc.load_gather(w_blk, [ix])
            pltpu.make_async_copy(x_hbm.at[src_blk.at[pl.ds(off, _K)]],
                                  rows_b, sem_b).wait()

            def scale(r, _):
                wv = plsc.load_gather(wC, [jnp.full((16,), r, jnp.int32)])
                for j in range(_D // 16):
                    rows_b[r, pl.ds(j * 16, 16)] = (
                        rows_b[r, pl.ds(j * 16, 16)] * wv)
                return 0

            lax.fori_loop(0, _K, scale, 0, unroll=2)
            pltpu.sync_copy(rows_b, acc_sh.at[dstC], add=True)

            if second:
                pltpu.make_async_copy(inv_hbm.at[dst_blk.at[pl.ds(off, _K)]],
                                      val_b, sem2_b).wait()
                for t in range(_K // 16):
                    srcC[pl.ds(t * 16, 16)] = plsc.load_gather(
                        src_blk, [off + t * 16 + iota])
                    val_b[pl.ds(t * 16, 16)] = (
                        val_b[pl.ds(t * 16, 16)] * wC[pl.ds(t * 16, 16)])
                pltpu.sync_copy(val_b, sca_sh.at[srcC], add=True)
            else:
                pltpu.sync_copy(wC, sca_sh.at[dstC], add=True)

        assert _CPB % 2 == 1
        for b in range(_NBLK):
            base = ebase + b * _B
            pltpu.sync_copy(src_hbm.at[pl.ds(base, _B)], src_blk)
            pltpu.sync_copy(dst_hbm.at[pl.ds(base, _B)], dst_blk)
            pltpu.sync_copy(w_hbm.at[pl.ds(base, _B)], w_blk)

            issue(0, *bufA)

            def pair(p, _):
                issue((2 * p + 1) * _K, *bufB)
                process((2 * p) * _K, *bufA)
                issue((2 * p + 2) * _K, *bufA)
                process((2 * p + 1) * _K, *bufB)
                return 0

            lax.fori_loop(0, _CPB // 2, pair, 0, unroll=False)
            process((_CPB - 1) * _K, *bufA)

        plsc.subcore_barrier()

        pltpu.sync_copy(acc_sh.at[pl.ds(s * _RPT, _RPT)],
                        num_out.at[c, pl.ds(s * _RPT, _RPT)])
        pltpu.sync_copy(sca_sh.at[pl.ds(s * _RPT, _RPT)],
                        sca_out.at[c, pl.ds(s * _RPT, _RPT)])

    return pl.kernel(body, out_type=out_type, mesh=mesh,
                     scratch_types=scratch,
                     compiler_params=pltpu.CompilerParams(
                         needs_layout_passes=False))


_sc_pass1 = _sc_edge_pass(second=False)
_sc_pass2 = _sc_edge_pass(second=True)


def _tc0_body(num_ref, den_ref, x_ref, wl_ref, wr_ref, b_ref, h1_ref, inv_ref):
    den = den_ref[0, :_N] + den_ref[1, :_N]
    inv = 1.0 / jnp.clip(den, 1e-12, None)
    num = num_ref[0, :_N] + num_ref[1, :_N]
    mean = num * inv[:, None]
    h = (jnp.dot(mean, wl_ref[...], preferred_element_type=jnp.float32)
         + jnp.dot(x_ref[...], wr_ref[...], preferred_element_type=jnp.float32)
         + b_ref[...][None, :])
    h1_ref[...] = jnp.maximum(h, 0.0)
    inv_ref[...] = inv


_tc_layer0 = pl.pallas_call(
    _tc0_body,
    out_shape=(
        jax.ShapeDtypeStruct((_N, _D), jnp.float32),
        jax.ShapeDtypeStruct((_N,), jnp.float32),
    ),
)


def _tc12_body(num_ref, inv_ref, h1_ref, c_ref, wl1_ref, wr1_ref, b1_ref,
               wl2_ref, wr2_ref, b2_ref, out_ref):
    inv = inv_ref[...]
    num = num_ref[0, :_N] + num_ref[1, :_N]
    mean = num * inv[:, None]
    h2 = (jnp.dot(mean, wl1_ref[...], preferred_element_type=jnp.float32)
          + jnp.dot(h1_ref[...], wr1_ref[...], preferred_element_type=jnp.float32)
          + b1_ref[...][None, :])
    h2 = jnp.maximum(h2, 0.0)
    cvec = (c_ref[0, :_N] + c_ref[1, :_N]) * (1.0 / _N)
    s_c = jnp.dot(cvec[None, :], h2, preferred_element_type=jnp.float32)
    s_m = jnp.sum(h2, axis=0, keepdims=True) * (1.0 / _N)
    out = (jnp.dot(s_c, wl2_ref[...], preferred_element_type=jnp.float32)
           + jnp.dot(s_m, wr2_ref[...], preferred_element_type=jnp.float32)
           + b2_ref[...][None, :])
    out_ref[...] = out


_tc_layer12 = pl.pallas_call(
    _tc12_body,
    out_shape=jax.ShapeDtypeStruct((1, 16), jnp.float32),
)


@jax.jit
def kernel(x, edge_index, edge_attr, Wl0, Wr0, b0, Wl1, Wr1, b1, Wl2, Wr2, b2):
    src = edge_index[0]
    dst = edge_index[1]
    w = edge_attr

    num0p, den0p = _sc_pass1(x, src, dst, w)
    h1, inv_den = _tc_layer0(num0p, den0p, x, Wl0, Wr0, b0)
    num1p, cp = _sc_pass2(h1, src, dst, w, inv_den)
    out = _tc_layer12(num1p, inv_den, h1, cp, Wl1, Wr1, b1, Wl2, Wr2, b2)
    return out[0]


# speedup vs baseline: 12.9332x; 1.0032x over previous
import functools

import jax
import jax.numpy as jnp
from jax import lax
from jax.experimental import pallas as pl
from jax.experimental.pallas import tpu as pltpu
from jax.experimental.pallas import tpu_sc as plsc

_N = 10000
_E = 320000
_D = 128
_NC = 2
_NS = 16
_NW = _NC * _NS
_EPW = _E // _NW
_K = 80
_B = 2000
_CPB = _B // _K
_NBLK = _EPW // _B
_NP = 10240
_RPT = _NP // _NS


def _zero_rows(ref, nrows, width16):
    z = jnp.zeros((16,), jnp.float32)

    def body(r, _):
        for j in range(width16):
            ref[r, pl.ds(j * 16, 16)] = z
        return 0

    lax.fori_loop(0, nrows, body, 0, unroll=False)


def _sc_edge_pass(second):
    mesh = plsc.VectorSubcoreMesh(core_axis_name="c", subcore_axis_name="s")

    out_type = (
        jax.ShapeDtypeStruct((_NC, _NP, _D), jnp.float32),
        jax.ShapeDtypeStruct((_NC, _NP), jnp.float32),
    )
    buf = lambda dt: pltpu.VMEM((_K,), dt)
    scratch = [
        pltpu.VMEM((_B,), jnp.int32),
        pltpu.VMEM((_B,), jnp.int32),
        pltpu.VMEM((_B,), jnp.float32),
        buf(jnp.int32),
        buf(jnp.int32),
        buf(jnp.float32),
        pltpu.VMEM((_K, _D), jnp.float32),
        pltpu.VMEM((_K, _D), jnp.float32),
        buf(jnp.float32), buf(jnp.float32),
        pltpu.VMEM((_RPT,), jnp.float32),
        pltpu.VMEM_SHARED((_NP, _D), jnp.float32),
        pltpu.VMEM_SHARED((_NP,), jnp.float32),
        pltpu.SemaphoreType.DMA, pltpu.SemaphoreType.DMA,
        pltpu.SemaphoreType.DMA, pltpu.SemaphoreType.DMA,
    ]

    def body(*refs):
        if second:
            (x_hbm, src_hbm, dst_hbm, w_hbm, inv_hbm,
             num_out, sca_out,
             src_blk, dst_blk, w_blk, dstC, srcC, wC, rowsA, rowsB,
             valA, valB, zvec_v, acc_sh, sca_sh,
             semA, semB, sem2A, sem2B) = refs
        else:
            (x_hbm, src_hbm, dst_hbm, w_hbm,
             num_out, sca_out,
             src_blk, dst_blk, w_blk, dstC, srcC, wC, rowsA, rowsB,
             valA, valB, zvec_v, acc_sh, sca_sh,
             semA, semB, sem2A, sem2B) = refs

        c = lax.axis_index("c")
        s = lax.axis_index("s")
        wid = s * _NC + c
        iota = lax.iota(jnp.int32, 16)

        _zero_rows(rowsA, _K, _D // 16)
        for q in range(_RPT // _K):
            pltpu.sync_copy(rowsA, acc_sh.at[pl.ds(s * _RPT + q * _K, _K)])

        z16 = jnp.zeros((16,), jnp.float32)

        def zvec_body(i, _):
            zvec_v[pl.ds(i * 16, 16)] = z16
            return 0

        lax.fori_loop(0, _RPT // 16, zvec_body, 0, unroll=False)
        pltpu.sync_copy(zvec_v, sca_sh.at[pl.ds(s * _RPT, _RPT)])

        plsc.subcore_barrier()

        ebase = wid * _EPW
        bufA = (rowsA, valA, semA, sem2A)
        bufB = (rowsB, valB, semB, sem2B)

        def _align8(off):
            return off if isinstance(off, int) else pl.multiple_of(off, 8)

        def issue(off, rows_b, val_b, sem_b, sem2_b):
            off = _align8(off)
            pltpu.async_copy(
                x_hbm.at[src_blk.at[pl.ds(off, _K)]], rows_b, sem_b)
            if second:
                pltpu.async_copy(
                    inv_hbm.at[dst_blk.at[pl.ds(off, _K)]], val_b, sem2_b)

        def process(off, rows_b, val_b, sem_b, sem2_b):
            off = _align8(off)
            for j in range(_K // 16):
                ix = off + j * 16 + iota
                dstC[pl.ds(j * 16, 16)] = plsc.load_gather(dst_blk, [ix])
                wC[pl.ds(j * 16, 16)] = plsc.load_gather(w_blk, [ix])
            pltpu.make_async_copy(x_hbm.at[src_blk.at[pl.ds(off, _K)]],
                                  rows_b, sem_b).wait()

            def scale(r, _):
                wv = plsc.load_gather(wC, [jnp.full((16,), r, jnp.int32)])
                for j in range(_D // 16):
                    rows_b[r, pl.ds(j * 16, 16)] = (
                        rows_b[r, pl.ds(j * 16, 16)] * wv)
                return 0

            lax.fori_loop(0, _K, scale, 0, unroll=4)
            pltpu.sync_copy(rows_b, acc_sh.at[dstC], add=True)

            if second:
                pltpu.make_async_copy(inv_hbm.at[dst_blk.at[pl.ds(off, _K)]],
                                      val_b, sem2_b).wait()
                for t in range(_K // 16):
                    srcC[pl.ds(t * 16, 16)] = plsc.load_gather(
                        src_blk, [off + t * 16 + iota])
                    val_b[pl.ds(t * 16, 16)] = (
                        val_b[pl.ds(t * 16, 16)] * wC[pl.ds(t * 16, 16)])
                pltpu.sync_copy(val_b, sca_sh.at[srcC], add=True)
            else:
                pltpu.sync_copy(wC, sca_sh.at[dstC], add=True)

        assert _CPB % 2 == 1
        for b in range(_NBLK):
            base = ebase + b * _B
            pltpu.sync_copy(src_hbm.at[pl.ds(base, _B)], src_blk)
            pltpu.sync_copy(dst_hbm.at[pl.ds(base, _B)], dst_blk)
            pltpu.sync_copy(w_hbm.at[pl.ds(base, _B)], w_blk)

            issue(0, *bufA)

            def pair(p, _):
                issue((2 * p + 1) * _K, *bufB)
                process((2 * p) * _K, *bufA)
                issue((2 * p + 2) * _K, *bufA)
                process((2 * p + 1) * _K, *bufB)
                return 0

            lax.fori_loop(0, _CPB // 2, pair, 0, unroll=False)
            process((_CPB - 1) * _K, *bufA)

        plsc.subcore_barrier()

        pltpu.sync_copy(acc_sh.at[pl.ds(s * _RPT, _RPT)],
                        num_out.at[c, pl.ds(s * _RPT, _RPT)])
        pltpu.sync_copy(sca_sh.at[pl.ds(s * _RPT, _RPT)],
                        sca_out.at[c, pl.ds(s * _RPT, _RPT)])

    return pl.kernel(body, out_type=out_type, mesh=mesh,
                     scratch_types=scratch,
                     compiler_params=pltpu.CompilerParams(
                         needs_layout_passes=False))


_sc_pass1 = _sc_edge_pass(second=False)
_sc_pass2 = _sc_edge_pass(second=True)


def _tc0_body(num_ref, den_ref, x_ref, wl_ref, wr_ref, b_ref, h1_ref, inv_ref):
    den = den_ref[0, :_N] + den_ref[1, :_N]
    inv = 1.0 / jnp.clip(den, 1e-12, None)
    num = num_ref[0, :_N] + num_ref[1, :_N]
    mean = num * inv[:, None]
    h = (jnp.dot(mean, wl_ref[...], preferred_element_type=jnp.float32)
         + jnp.dot(x_ref[...], wr_ref[...], preferred_element_type=jnp.float32)
         + b_ref[...][None, :])
    h1_ref[...] = jnp.maximum(h, 0.0)
    inv_ref[...] = inv


_tc_layer0 = pl.pallas_call(
    _tc0_body,
    out_shape=(
        jax.ShapeDtypeStruct((_N, _D), jnp.float32),
        jax.ShapeDtypeStruct((_N,), jnp.float32),
    ),
)


def _tc12_body(num_ref, inv_ref, h1_ref, c_ref, wl1_ref, wr1_ref, b1_ref,
               wl2_ref, wr2_ref, b2_ref, out_ref):
    inv = inv_ref[...]
    num = num_ref[0, :_N] + num_ref[1, :_N]
    mean = num * inv[:, None]
    h2 = (jnp.dot(mean, wl1_ref[...], preferred_element_type=jnp.float32)
          + jnp.dot(h1_ref[...], wr1_ref[...], preferred_element_type=jnp.float32)
          + b1_ref[...][None, :])
    h2 = jnp.maximum(h2, 0.0)
    cvec = (c_ref[0, :_N] + c_ref[1, :_N]) * (1.0 / _N)
    s_c = jnp.dot(cvec[None, :], h2, preferred_element_type=jnp.float32)
    s_m = jnp.sum(h2, axis=0, keepdims=True) * (1.0 / _N)
    out = (jnp.dot(s_c, wl2_ref[...], preferred_element_type=jnp.float32)
           + jnp.dot(s_m, wr2_ref[...], preferred_element_type=jnp.float32)
           + b2_ref[...][None, :])
    out_ref[...] = out


_tc_layer12 = pl.pallas_call(
    _tc12_body,
    out_shape=jax.ShapeDtypeStruct((1, 16), jnp.float32),
)


@jax.jit
def kernel(x, edge_index, edge_attr, Wl0, Wr0, b0, Wl1, Wr1, b1, Wl2, Wr2, b2):
    src = edge_index[0]
    dst = edge_index[1]
    w = edge_attr

    num0p, den0p = _sc_pass1(x, src, dst, w)
    h1, inv_den = _tc_layer0(num0p, den0p, x, Wl0, Wr0, b0)
    num1p, cp = _sc_pass2(h1, src, dst, w, inv_den)
    out = _tc_layer12(num1p, inv_den, h1, cp, Wl1, Wr1, b1, Wl2, Wr2, b2)
    return out[0]
